# baseline (speedup 1.0000x reference)
import jax
import jax.numpy as jnp
from jax.experimental import pallas as pl
from jax.experimental.pallas import tpu as pltpu

N = 5000
D = 12544
H = 1024
BM = 200
BK = 1792
KBLKS = D // BK
MBLKS = N // BM


def _body(x_ref, w1_ref, b1_ref, w2_ref, b2_ref, w3_ref, b3_ref,
          w4_ref, b4_ref, cls_ref, box_ref, acc_ref):
    k = pl.program_id(0)
    i = pl.program_id(1)
    rows = pl.ds(i * BM, BM)

    part = jnp.dot(x_ref[...].astype(jnp.bfloat16),
                   w1_ref[...].astype(jnp.bfloat16),
                   preferred_element_type=jnp.float32)

    @pl.when(k == 0)
    def _():
        acc_ref[rows, :] = part

    @pl.when(k > 0)
    def _():
        acc_ref[rows, :] += part

    @pl.when(k == KBLKS - 1)
    def _():
        h1 = jnp.maximum(acc_ref[rows, :] + b1_ref[...], 0.0
                         ).astype(jnp.bfloat16)
        h2 = jnp.maximum(
            jnp.dot(h1, w2_ref[...].astype(jnp.bfloat16),
                    preferred_element_type=jnp.float32)
            + b2_ref[...], 0.0).astype(jnp.bfloat16)
        logits = jnp.dot(h2, w3_ref[...].astype(jnp.bfloat16),
                         preferred_element_type=jnp.float32) + b3_ref[...]
        m = jnp.max(logits, axis=-1, keepdims=True)
        e = jnp.exp(logits - m)
        cls_ref[...] = e / jnp.sum(e, axis=-1, keepdims=True)
        box_ref[...] = jnp.dot(h2, w4_ref[...].astype(jnp.bfloat16),
                               preferred_element_type=jnp.float32) + b4_ref[...]


def kernel(feature_vectors, W1, b1, W2, b2, W3, b3, W4, b4):
    C1 = W3.shape[1]
    C4 = W4.shape[1]
    grid = (KBLKS, MBLKS)
    out = pl.pallas_call(
        _body,
        grid=grid,
        in_specs=[
            pl.BlockSpec((BM, BK), lambda k, i: (i, k)),
            pl.BlockSpec((BK, H), lambda k, i: (k, 0)),
            pl.BlockSpec((1, H), lambda k, i: (0, 0)),
            pl.BlockSpec((H, H), lambda k, i: (0, 0)),
            pl.BlockSpec((1, H), lambda k, i: (0, 0)),
            pl.BlockSpec((H, C1), lambda k, i: (0, 0)),
            pl.BlockSpec((1, C1), lambda k, i: (0, 0)),
            pl.BlockSpec((H, C4), lambda k, i: (0, 0)),
            pl.BlockSpec((1, C4), lambda k, i: (0, 0)),
        ],
        out_specs=[
            pl.BlockSpec((BM, C1), lambda k, i: (i, 0)),
            pl.BlockSpec((BM, C4), lambda k, i: (i, 0)),
        ],
        out_shape=[
            jax.ShapeDtypeStruct((N, C1), jnp.float32),
            jax.ShapeDtypeStruct((N, C4), jnp.float32),
        ],
        scratch_shapes=[pltpu.VMEM((N, H), jnp.float32)],
        compiler_params=pltpu.CompilerParams(
            dimension_semantics=("arbitrary", "arbitrary"),
        ),
    )(feature_vectors, W1, b1.reshape(1, H), W2, b2.reshape(1, H),
      W3, b3.reshape(1, C1), W4, b4.reshape(1, C4))
    return (out[0], out[1])


# speedup vs baseline: 1.1881x; 1.1881x over previous
import jax
import jax.numpy as jnp
from jax.experimental import pallas as pl
from jax.experimental.pallas import tpu as pltpu

N = 5000
D = 12544
H = 1024
BM = 1000
BK = 896
KBLKS = D // BK
MBLKS = N // BM


def _body(x_ref, w1_ref, b1_ref, w2_ref, b2_ref, w3_ref, b3_ref,
          w4_ref, b4_ref, cls_ref, box_ref, acc_ref):
    k = pl.program_id(0)
    i = pl.program_id(1)
    rows = pl.ds(i * BM, BM)

    part = jnp.dot(x_ref[...].astype(jnp.bfloat16),
                   w1_ref[...].astype(jnp.bfloat16),
                   preferred_element_type=jnp.float32)

    @pl.when(k == 0)
    def _():
        acc_ref[rows, :] = part

    @pl.when(k > 0)
    def _():
        acc_ref[rows, :] += part

    @pl.when(k == KBLKS - 1)
    def _():
        h1 = jnp.maximum(acc_ref[rows, :] + b1_ref[...], 0.0
                         ).astype(jnp.bfloat16)
        h2 = jnp.maximum(
            jnp.dot(h1, w2_ref[...].astype(jnp.bfloat16),
                    preferred_element_type=jnp.float32)
            + b2_ref[...], 0.0).astype(jnp.bfloat16)
        logits = jnp.dot(h2, w3_ref[...].astype(jnp.bfloat16),
                         preferred_element_type=jnp.float32) + b3_ref[...]
        m = jnp.max(logits, axis=-1, keepdims=True)
        e = jnp.exp(logits - m)
        cls_ref[...] = e / jnp.sum(e, axis=-1, keepdims=True)
        box_ref[...] = jnp.dot(h2, w4_ref[...].astype(jnp.bfloat16),
                               preferred_element_type=jnp.float32) + b4_ref[...]


def kernel(feature_vectors, W1, b1, W2, b2, W3, b3, W4, b4):
    C1 = W3.shape[1]
    C4 = W4.shape[1]
    grid = (KBLKS, MBLKS)
    out = pl.pallas_call(
        _body,
        grid=grid,
        in_specs=[
            pl.BlockSpec((BM, BK), lambda k, i: (i, k)),
            pl.BlockSpec((BK, H), lambda k, i: (k, 0)),
            pl.BlockSpec((1, H), lambda k, i: (0, 0)),
            pl.BlockSpec((H, H), lambda k, i: (0, 0)),
            pl.BlockSpec((1, H), lambda k, i: (0, 0)),
            pl.BlockSpec((H, C1), lambda k, i: (0, 0)),
            pl.BlockSpec((1, C1), lambda k, i: (0, 0)),
            pl.BlockSpec((H, C4), lambda k, i: (0, 0)),
            pl.BlockSpec((1, C4), lambda k, i: (0, 0)),
        ],
        out_specs=[
            pl.BlockSpec((BM, C1), lambda k, i: (i, 0)),
            pl.BlockSpec((BM, C4), lambda k, i: (i, 0)),
        ],
        out_shape=[
            jax.ShapeDtypeStruct((N, C1), jnp.float32),
            jax.ShapeDtypeStruct((N, C4), jnp.float32),
        ],
        scratch_shapes=[pltpu.VMEM((N, H), jnp.float32)],
        compiler_params=pltpu.CompilerParams(
            dimension_semantics=("arbitrary", "arbitrary"),
        ),
    )(feature_vectors, W1, b1.reshape(1, H), W2, b2.reshape(1, H),
      W3, b3.reshape(1, C1), W4, b4.reshape(1, C4))
    return (out[0], out[1])


# speedup vs baseline: 1.4519x; 1.2220x over previous
import jax
import jax.numpy as jnp
from jax.experimental import pallas as pl
from jax.experimental.pallas import tpu as pltpu

N = 5000
D = 12544
H = 1024
BM = 1000
BK = 1792
KBLKS = D // BK
MBLKS = N // BM


def _body(x_ref, w1_ref, b1_ref, w2_ref, b2_ref, w3_ref, b3_ref,
          w4_ref, b4_ref, cls_ref, box_ref, acc_ref):
    k = pl.program_id(0)
    i = pl.program_id(1)
    rows = pl.ds(i * BM, BM)

    part = jnp.dot(x_ref[...].astype(jnp.bfloat16),
                   w1_ref[...].astype(jnp.bfloat16),
                   preferred_element_type=jnp.float32)

    @pl.when(k == 0)
    def _():
        acc_ref[rows, :] = part

    @pl.when(k > 0)
    def _():
        acc_ref[rows, :] += part

    @pl.when(k == KBLKS - 1)
    def _():
        h1 = jnp.maximum(acc_ref[rows, :] + b1_ref[...], 0.0
                         ).astype(jnp.bfloat16)
        h2 = jnp.maximum(
            jnp.dot(h1, w2_ref[...].astype(jnp.bfloat16),
                    preferred_element_type=jnp.float32)
            + b2_ref[...], 0.0).astype(jnp.bfloat16)
        logits = jnp.dot(h2, w3_ref[...].astype(jnp.bfloat16),
                         preferred_element_type=jnp.float32) + b3_ref[...]
        m = jnp.max(logits, axis=-1, keepdims=True)
        e = jnp.exp(logits - m)
        cls_ref[...] = e / jnp.sum(e, axis=-1, keepdims=True)
        box_ref[...] = jnp.dot(h2, w4_ref[...].astype(jnp.bfloat16),
                               preferred_element_type=jnp.float32) + b4_ref[...]


def kernel(feature_vectors, W1, b1, W2, b2, W3, b3, W4, b4):
    C1 = W3.shape[1]
    C4 = W4.shape[1]
    grid = (KBLKS, MBLKS)
    out = pl.pallas_call(
        _body,
        grid=grid,
        in_specs=[
            pl.BlockSpec((BM, BK), lambda k, i: (i, k)),
            pl.BlockSpec((BK, H), lambda k, i: (k, 0)),
            pl.BlockSpec((1, H), lambda k, i: (0, 0)),
            pl.BlockSpec((H, H), lambda k, i: (0, 0)),
            pl.BlockSpec((1, H), lambda k, i: (0, 0)),
            pl.BlockSpec((H, C1), lambda k, i: (0, 0)),
            pl.BlockSpec((1, C1), lambda k, i: (0, 0)),
            pl.BlockSpec((H, C4), lambda k, i: (0, 0)),
            pl.BlockSpec((1, C4), lambda k, i: (0, 0)),
        ],
        out_specs=[
            pl.BlockSpec((BM, C1), lambda k, i: (i, 0)),
            pl.BlockSpec((BM, C4), lambda k, i: (i, 0)),
        ],
        out_shape=[
            jax.ShapeDtypeStruct((N, C1), jnp.float32),
            jax.ShapeDtypeStruct((N, C4), jnp.float32),
        ],
        scratch_shapes=[pltpu.VMEM((N, H), jnp.float32)],
        compiler_params=pltpu.CompilerParams(
            dimension_semantics=("arbitrary", "arbitrary"),
            vmem_limit_bytes=64 * 1024 * 1024,
        ),
    )(feature_vectors, W1, b1.reshape(1, H), W2, b2.reshape(1, H),
      W3, b3.reshape(1, C1), W4, b4.reshape(1, C4))
    return (out[0], out[1])
